# Initial kernel scaffold; baseline (speedup 1.0000x reference)
import jax
import jax.numpy as jnp
from jax.experimental import pallas as pl


def kernel(logits):
    raise NotImplementedError("write your pallas kernel here")


# speedup vs baseline: 1.0161x; 1.0161x over previous
import functools

import jax
import jax.numpy as jnp
import numpy as np
from jax import lax
from jax.experimental import pallas as pl

N_SAMPLES = 128

_ROT_A = (13, 15, 26, 6)
_ROT_B = (17, 29, 16, 24)
_KS = (0, 42, (0 ^ 42 ^ 0x1BD11BDA))

_TINY = np.float32(np.finfo(np.float32).tiny)
_ONE_BITS = np.int32(0x3F800000)


def _threefry_bits(cnt_lo):
    x0 = jnp.zeros_like(cnt_lo) + np.int32(_KS[0])
    x1 = cnt_lo + np.int32(_KS[1])
    for i in range(5):
        rots = _ROT_A if i % 2 == 0 else _ROT_B
        for r in rots:
            x0 = x0 + x1
            x1 = lax.shift_left(x1, np.int32(r)) | lax.shift_right_logical(
                x1, np.int32(32 - r))
            x1 = x0 ^ x1
        x0 = x0 + np.int32(_KS[(i + 1) % 3])
        x1 = x1 + np.int32(_KS[(i + 2) % 3] + (i + 1))
    return x0 ^ x1


def _body(logits_ref, out_ref, *, bn, bl, l, c):
    pn = pl.program_id(0)
    pidl = pl.program_id(1)
    base = pn * (bn * l * c) + pidl * (bl * c)

    shape = (bn, bl, c)
    i_n = lax.broadcasted_iota(jnp.int32, shape, 0)
    i_l = lax.broadcasted_iota(jnp.int32, shape, 1)
    lane = lax.broadcasted_iota(jnp.int32, shape, 2)
    cnt = base + i_n * (l * c) + i_l * c + lane

    bits = _threefry_bits(cnt)

    fb = lax.shift_right_logical(bits, np.int32(9)) | _ONE_BITS
    u0 = lax.bitcast_convert_type(fb, jnp.float32) - np.float32(1.0)
    u = jnp.maximum(_TINY, u0 + _TINY)

    g = -jnp.log(-jnp.log(u))
    v = g + logits_ref[0][None, :, :]

    m = jnp.max(v, axis=2, keepdims=True)
    idx = jnp.min(jnp.where(v == m, lane, np.int32(c)), axis=2, keepdims=True)
    out_ref[...] = (lane == idx).astype(jnp.float32)


def kernel(logits):
    _, l, c = logits.shape
    n = N_SAMPLES
    bn, bl = 16, 128
    body = functools.partial(_body, bn=bn, bl=bl, l=l, c=c)
    return pl.pallas_call(
        body,
        grid=(n // bn, l // bl),
        in_specs=[pl.BlockSpec((1, bl, c), lambda pn, pidl: (0, pidl, 0))],
        out_specs=pl.BlockSpec((bn, bl, c), lambda pn, pidl: (pn, pidl, 0)),
        out_shape=jax.ShapeDtypeStruct((n, l, c), jnp.float32),
    )(logits)
